# baseline (speedup 1.0000x reference)
import functools

import jax
import jax.numpy as jnp
import numpy as np
from jax import lax
from jax.experimental import pallas as pl
from jax.experimental.pallas import tpu as pltpu
from jax.experimental.pallas import tpu_sc as plsc

B = 512
NN = 16
N = B * NN
EDGES_PER_G = NN * NN
E = B * EDGES_PER_G
D = 16
HID = 256
FEAT = 128
NUM_ATOMS = 9

_NC = 2
_NS = 16
_NW = _NC * _NS
_BPW = N // _NW

G_BLK = 32
NODES_BLK = G_BLK * NN
ROWS_BLK = G_BLK * EDGES_PER_G
SEL_K = 2 * NODES_BLK + G_BLK


def _build_sel() -> np.ndarray:
    e = np.arange(ROWS_BLK)
    g, r = e // EDGES_PER_G, e % EDGES_PER_G
    a, b = r // NN, r % NN
    n = np.arange(NODES_BLK)
    sel_a = (n[None, :] == (NN * g + a)[:, None])
    sel_b = (n[None, :] == (NN * g + b)[:, None])
    sel_g = (np.arange(G_BLK)[None, :] == g[:, None])
    return np.concatenate([sel_a, sel_b, sel_g], axis=1).astype(np.float32)


_SEL = _build_sel()


def _make_sc_gather_body(bpw):
    def _sc_gather_body(idx_hbm, mult_hbm, idtab_hbm, multtab_hbm,
                        xid_out, xmult_out, idx_v, rows_v, sem):
        wid = lax.axis_index("s") * _NC + lax.axis_index("c")
        base = wid * bpw
        pltpu.sync_copy(idx_hbm.at[pl.ds(base, bpw)], idx_v)
        pltpu.async_copy(idtab_hbm.at[idx_v], rows_v, sem).wait()
        pltpu.sync_copy(rows_v, xid_out.at[pl.ds(base, bpw)])
        pltpu.sync_copy(mult_hbm.at[pl.ds(base, bpw)], idx_v)
        pltpu.async_copy(multtab_hbm.at[idx_v], rows_v, sem).wait()
        pltpu.sync_copy(rows_v, xmult_out.at[pl.ds(base, bpw)])
    return _sc_gather_body


@jax.jit
def _sc_gather(idx, mult, id_table, mult_table):
    n = idx.shape[0]
    bpw = n // _NW
    mesh = plsc.VectorSubcoreMesh(core_axis_name="c", subcore_axis_name="s")
    fn = functools.partial(
        pl.kernel,
        out_type=[
            jax.ShapeDtypeStruct((n, D), jnp.float32),
            jax.ShapeDtypeStruct((n, D), jnp.float32),
        ],
        mesh=mesh,
        scratch_types=[
            pltpu.VMEM((bpw,), jnp.int32),
            pltpu.VMEM((bpw, D), jnp.float32),
            pltpu.SemaphoreType.DMA,
        ],
        compiler_params=pltpu.CompilerParams(use_tc_tiling_on_sc=False),
    )(_make_sc_gather_body(bpw))
    return fn(idx, mult, id_table, mult_table)


def _tc_mlp_body(xid_ref, xm_ref, z_ref, w1_ref, b1_ref, w2_ref,
                 b2_ref, w3_ref, b3_ref, out_ref):
    f32 = jnp.float32
    xid = xid_ref[...]
    xm = xm_ref[...]
    w1 = w1_ref[...]
    dot = functools.partial(jnp.dot, preferred_element_type=f32)
    xcat = jnp.concatenate([xid, xm], axis=1)
    wcat = jnp.concatenate([w1[0:32], w1[32:64], w1[96:128]], axis=1)
    big = dot(xcat, wcat)
    xs = big[:, 0:HID]
    xd = big[:, HID:2 * HID]
    xa = big[:, 2 * HID:3 * HID]
    gi = lax.broadcasted_iota(jnp.int32, (G_BLK, NODES_BLK), 0)
    ni = lax.broadcasted_iota(jnp.int32, (G_BLK, NODES_BLK), 1)
    ones_sel = (ni // NN == gi).astype(f32)
    c = (dot(ones_sel, xa) + dot(z_ref[...], w1[64:96])
         + b1_ref[...])
    bf16 = jnp.bfloat16
    xs3 = xs.astype(bf16).reshape(G_BLK, NN, HID)
    xd3 = xd.astype(bf16).reshape(G_BLK, NN, HID)
    cb = c.astype(bf16)
    h1 = jax.nn.relu(xs3[:, :, None, :] + xd3[:, None, :, :]
                     + cb[:, None, None, :])
    h1 = h1.reshape(ROWS_BLK, HID)
    h2 = jax.nn.relu(dot(h1, w2_ref[...]).astype(bf16)
                     + b2_ref[...])
    o = dot(h2, w3_ref[...]) + b3_ref[...]
    r = lax.broadcasted_iota(jnp.int32, (EDGES_PER_G, EDGES_PER_G), 0)
    cc = lax.broadcasted_iota(jnp.int32, (EDGES_PER_G, EDGES_PER_G), 1)
    Q = 0.5 * ((cc == (r % NN) * NN + r // NN).astype(f32)
               + (cc == r).astype(f32))
    cols = []
    for g in range(G_BLK):
        og = o[g * EDGES_PER_G:(g + 1) * EDGES_PER_G]
        cols.append(lax.dot_general(
            og, Q, (((0,), (0,)), ((), ())),
            preferred_element_type=f32))
    out_ref[...] = jnp.concatenate(cols, axis=1)


def _tc_mlp(xid, xmult, z_graph, W1, b1, W2, b2, W3, b3, interpret=False):
    nb = z_graph.shape[0]
    grid = nb // G_BLK
    return pl.pallas_call(
        _tc_mlp_body,
        grid=(grid,),
        in_specs=[
            pl.BlockSpec((NODES_BLK, D), lambda i: (i, 0)),
            pl.BlockSpec((NODES_BLK, D), lambda i: (i, 0)),
            pl.BlockSpec((G_BLK, 32), lambda i: (i, 0)),
            pl.BlockSpec((FEAT, HID), lambda i: (0, 0)),
            pl.BlockSpec((1, HID), lambda i: (0, 0)),
            pl.BlockSpec((HID, FEAT), lambda i: (0, 0)),
            pl.BlockSpec((1, FEAT), lambda i: (0, 0)),
            pl.BlockSpec((FEAT, NUM_ATOMS), lambda i: (0, 0)),
            pl.BlockSpec((1, NUM_ATOMS), lambda i: (0, 0)),
        ],
        out_specs=pl.BlockSpec((NUM_ATOMS, ROWS_BLK), lambda i: (0, i)),
        out_shape=jax.ShapeDtypeStruct(
            (NUM_ATOMS, nb * EDGES_PER_G), jnp.float32),
        compiler_params=pltpu.CompilerParams(
            dimension_semantics=("arbitrary",)),
        interpret=interpret,
    )(xid, xmult, z_graph, W1, b1, W2, b2, W3, b3)


def kernel(shape_node_idx, shape_node_mult, z_graph, id_table, mult_table,
           W1, b1, W2, b2, W3, b3):
    idx = shape_node_idx.astype(jnp.int32)
    mult = shape_node_mult.astype(jnp.int32)
    nh = N // 2
    gh = B // 2
    halves = []
    gathered = [_sc_gather(idx[h * nh:(h + 1) * nh],
                           mult[h * nh:(h + 1) * nh],
                           id_table, mult_table) for h in range(2)]
    for h in range(2):
        xid, xmult = gathered[h]
        halves.append(_tc_mlp(
            xid, xmult, z_graph[h * gh:(h + 1) * gh], W1,
            b1.reshape(1, HID), W2.astype(jnp.bfloat16),
            b2.astype(jnp.bfloat16).reshape(1, FEAT),
            W3.astype(jnp.bfloat16), b3.reshape(1, NUM_ATOMS)))
    return jnp.transpose(jnp.concatenate(halves, axis=1))


# speedup vs baseline: 1.1595x; 1.1595x over previous
import functools

import jax
import jax.numpy as jnp
import numpy as np
from jax import lax
from jax.experimental import pallas as pl
from jax.experimental.pallas import tpu as pltpu
from jax.experimental.pallas import tpu_sc as plsc

B = 512
NN = 16
N = B * NN
EDGES_PER_G = NN * NN
E = B * EDGES_PER_G
D = 16
HID = 256
FEAT = 128
NUM_ATOMS = 9

_NC = 2
_NS = 16
_NW = _NC * _NS
_BPW = N // _NW

G_BLK = 32
NODES_BLK = G_BLK * NN
ROWS_BLK = G_BLK * EDGES_PER_G
SEL_K = 2 * NODES_BLK + G_BLK


def _build_sel() -> np.ndarray:
    e = np.arange(ROWS_BLK)
    g, r = e // EDGES_PER_G, e % EDGES_PER_G
    a, b = r // NN, r % NN
    n = np.arange(NODES_BLK)
    sel_a = (n[None, :] == (NN * g + a)[:, None])
    sel_b = (n[None, :] == (NN * g + b)[:, None])
    sel_g = (np.arange(G_BLK)[None, :] == g[:, None])
    return np.concatenate([sel_a, sel_b, sel_g], axis=1).astype(np.float32)


_SEL = _build_sel()


def _make_sc_gather_body(bpw):
    def _sc_gather_body(idx_hbm, mult_hbm, idtab_hbm, multtab_hbm,
                        xid_out, xmult_out, idx_v, rows_v, sem):
        wid = lax.axis_index("s") * _NC + lax.axis_index("c")
        base = wid * bpw
        pltpu.sync_copy(idx_hbm.at[pl.ds(base, bpw)], idx_v)
        pltpu.async_copy(idtab_hbm.at[idx_v], rows_v, sem).wait()
        pltpu.sync_copy(rows_v, xid_out.at[pl.ds(base, bpw)])
        pltpu.sync_copy(mult_hbm.at[pl.ds(base, bpw)], idx_v)
        pltpu.async_copy(multtab_hbm.at[idx_v], rows_v, sem).wait()
        pltpu.sync_copy(rows_v, xmult_out.at[pl.ds(base, bpw)])
    return _sc_gather_body


@jax.jit
def _sc_gather(idx, mult, id_table, mult_table):
    n = idx.shape[0]
    bpw = n // _NW
    mesh = plsc.VectorSubcoreMesh(core_axis_name="c", subcore_axis_name="s")
    fn = functools.partial(
        pl.kernel,
        out_type=[
            jax.ShapeDtypeStruct((n, D), jnp.float32),
            jax.ShapeDtypeStruct((n, D), jnp.float32),
        ],
        mesh=mesh,
        scratch_types=[
            pltpu.VMEM((bpw,), jnp.int32),
            pltpu.VMEM((bpw, D), jnp.float32),
            pltpu.SemaphoreType.DMA,
        ],
        compiler_params=pltpu.CompilerParams(use_tc_tiling_on_sc=False),
    )(_make_sc_gather_body(bpw))
    return fn(idx, mult, id_table, mult_table)


def _tc_mlp_body(xid_ref, xm_ref, z_ref, w1_ref, b1_ref, w2_ref,
                 b2_ref, w3_ref, b3_ref, out_ref):
    f32 = jnp.float32
    xid = xid_ref[...]
    xm = xm_ref[...]
    w1 = w1_ref[...]
    dot = functools.partial(jnp.dot, preferred_element_type=f32)
    xcat = jnp.concatenate([xid, xm], axis=1)
    wcat = jnp.concatenate([w1[0:32], w1[32:64], w1[96:128]], axis=1)
    big = dot(xcat, wcat)
    xs = big[:, 0:HID]
    xd = big[:, HID:2 * HID]
    xa = big[:, 2 * HID:3 * HID]
    gi = lax.broadcasted_iota(jnp.int32, (G_BLK, NODES_BLK), 0)
    ni = lax.broadcasted_iota(jnp.int32, (G_BLK, NODES_BLK), 1)
    ones_sel = (ni // NN == gi).astype(f32)
    c = (dot(ones_sel, xa) + dot(z_ref[...], w1[64:96])
         + b1_ref[...])
    bf16 = jnp.bfloat16
    xs3 = xs.astype(bf16).reshape(G_BLK, NN, HID)
    xd3 = xd.astype(bf16).reshape(G_BLK, NN, HID)
    cb = c.astype(bf16)
    h1 = jax.nn.relu(xs3[:, :, None, :] + xd3[:, None, :, :]
                     + cb[:, None, None, :])
    h1 = h1.reshape(ROWS_BLK, HID)
    h2 = jax.nn.relu(dot(h1, w2_ref[...]).astype(bf16)
                     + b2_ref[...])
    o = dot(h2, w3_ref[...]) + b3_ref[...]
    r = lax.broadcasted_iota(jnp.int32, (EDGES_PER_G, EDGES_PER_G), 0)
    cc = lax.broadcasted_iota(jnp.int32, (EDGES_PER_G, EDGES_PER_G), 1)
    Q = 0.5 * ((cc == (r % NN) * NN + r // NN).astype(f32)
               + (cc == r).astype(f32))
    cols = []
    for g in range(G_BLK):
        og = o[g * EDGES_PER_G:(g + 1) * EDGES_PER_G]
        cols.append(lax.dot_general(
            og, Q, (((0,), (0,)), ((), ())),
            preferred_element_type=f32))
    out_ref[...] = jnp.concatenate(cols, axis=1)


def _tc_mlp(xid, xmult, z_graph, W1, b1, W2, b2, W3, b3, interpret=False):
    nb = z_graph.shape[0]
    grid = nb // G_BLK
    return pl.pallas_call(
        _tc_mlp_body,
        grid=(grid,),
        in_specs=[
            pl.BlockSpec((NODES_BLK, D), lambda i: (i, 0)),
            pl.BlockSpec((NODES_BLK, D), lambda i: (i, 0)),
            pl.BlockSpec((G_BLK, 32), lambda i: (i, 0)),
            pl.BlockSpec((FEAT, HID), lambda i: (0, 0)),
            pl.BlockSpec((1, HID), lambda i: (0, 0)),
            pl.BlockSpec((HID, FEAT), lambda i: (0, 0)),
            pl.BlockSpec((1, FEAT), lambda i: (0, 0)),
            pl.BlockSpec((FEAT, NUM_ATOMS), lambda i: (0, 0)),
            pl.BlockSpec((1, NUM_ATOMS), lambda i: (0, 0)),
        ],
        out_specs=pl.BlockSpec((NUM_ATOMS, ROWS_BLK), lambda i: (0, i)),
        out_shape=jax.ShapeDtypeStruct(
            (NUM_ATOMS, nb * EDGES_PER_G), jnp.float32),
        compiler_params=pltpu.CompilerParams(
            dimension_semantics=("arbitrary",)),
        interpret=interpret,
    )(xid, xmult, z_graph, W1, b1, W2, b2, W3, b3)


def kernel(shape_node_idx, shape_node_mult, z_graph, id_table, mult_table,
           W1, b1, W2, b2, W3, b3):
    idx = shape_node_idx.astype(jnp.int32)
    mult = shape_node_mult.astype(jnp.int32)
    xid, xmult = _sc_gather(idx, mult, id_table, mult_table)
    out_t = _tc_mlp(xid, xmult, z_graph, W1,
                    b1.reshape(1, HID), W2.astype(jnp.bfloat16),
                    b2.astype(jnp.bfloat16).reshape(1, FEAT),
                    W3.astype(jnp.bfloat16), b3.reshape(1, NUM_ATOMS))
    return jnp.transpose(out_t)


# speedup vs baseline: 1.1750x; 1.0134x over previous
import functools

import jax
import jax.numpy as jnp
import numpy as np
from jax import lax
from jax.experimental import pallas as pl
from jax.experimental.pallas import tpu as pltpu
from jax.experimental.pallas import tpu_sc as plsc

B = 512
NN = 16
N = B * NN
EDGES_PER_G = NN * NN
E = B * EDGES_PER_G
D = 16
HID = 256
FEAT = 128
NUM_ATOMS = 9

_NC = 2
_NS = 16
_NW = _NC * _NS
_BPW = N // _NW

G_BLK = 32
NODES_BLK = G_BLK * NN
ROWS_BLK = G_BLK * EDGES_PER_G
SEL_K = 2 * NODES_BLK + G_BLK


def _build_sel() -> np.ndarray:
    e = np.arange(ROWS_BLK)
    g, r = e // EDGES_PER_G, e % EDGES_PER_G
    a, b = r // NN, r % NN
    n = np.arange(NODES_BLK)
    sel_a = (n[None, :] == (NN * g + a)[:, None])
    sel_b = (n[None, :] == (NN * g + b)[:, None])
    sel_g = (np.arange(G_BLK)[None, :] == g[:, None])
    return np.concatenate([sel_a, sel_b, sel_g], axis=1).astype(np.float32)


_SEL = _build_sel()


def _make_sc_gather_body(bpw):
    def _sc_gather_body(idx_hbm, mult_hbm, idtab_hbm, multtab_hbm,
                        xid_out, xmult_out, idx_v, midx_v, rows_v, mrows_v,
                        sem, msem):
        wid = lax.axis_index("s") * _NC + lax.axis_index("c")
        base = wid * bpw
        pltpu.sync_copy(idx_hbm.at[pl.ds(base, bpw)], idx_v)
        pltpu.sync_copy(mult_hbm.at[pl.ds(base, bpw)], midx_v)
        c1 = pltpu.async_copy(idtab_hbm.at[idx_v], rows_v, sem)
        c2 = pltpu.async_copy(multtab_hbm.at[midx_v], mrows_v, msem)
        c1.wait()
        pltpu.sync_copy(rows_v, xid_out.at[pl.ds(base, bpw)])
        c2.wait()
        pltpu.sync_copy(mrows_v, xmult_out.at[pl.ds(base, bpw)])
    return _sc_gather_body


@jax.jit
def _sc_gather(idx, mult, id_table, mult_table):
    n = idx.shape[0]
    bpw = n // _NW
    mesh = plsc.VectorSubcoreMesh(core_axis_name="c", subcore_axis_name="s")
    fn = functools.partial(
        pl.kernel,
        out_type=[
            jax.ShapeDtypeStruct((n, D), jnp.float32),
            jax.ShapeDtypeStruct((n, D), jnp.float32),
        ],
        mesh=mesh,
        scratch_types=[
            pltpu.VMEM((bpw,), jnp.int32),
            pltpu.VMEM((bpw,), jnp.int32),
            pltpu.VMEM((bpw, D), jnp.float32),
            pltpu.VMEM((bpw, D), jnp.float32),
            pltpu.SemaphoreType.DMA,
            pltpu.SemaphoreType.DMA,
        ],
        compiler_params=pltpu.CompilerParams(use_tc_tiling_on_sc=False),
    )(_make_sc_gather_body(bpw))
    return fn(idx, mult, id_table, mult_table)


def _tc_mlp_body(xid_ref, xm_ref, z_ref, w1_ref, b1_ref, w2_ref,
                 b2_ref, w3_ref, b3_ref, out_ref):
    f32 = jnp.float32
    xid = xid_ref[...]
    xm = xm_ref[...]
    w1 = w1_ref[...]
    dot = functools.partial(jnp.dot, preferred_element_type=f32)
    xcat = jnp.concatenate([xid, xm], axis=1)
    wcat = jnp.concatenate([w1[0:32], w1[32:64], w1[96:128]], axis=1)
    big = dot(xcat, wcat)
    xs = big[:, 0:HID]
    xd = big[:, HID:2 * HID]
    xa = big[:, 2 * HID:3 * HID]
    gi = lax.broadcasted_iota(jnp.int32, (G_BLK, NODES_BLK), 0)
    ni = lax.broadcasted_iota(jnp.int32, (G_BLK, NODES_BLK), 1)
    ones_sel = (ni // NN == gi).astype(f32)
    c = (dot(ones_sel, xa) + dot(z_ref[...], w1[64:96])
         + b1_ref[...])
    bf16 = jnp.bfloat16
    xs3 = xs.astype(bf16).reshape(G_BLK, NN, HID)
    xd3 = xd.astype(bf16).reshape(G_BLK, NN, HID)
    cb = c.astype(bf16)
    h1 = jax.nn.relu(xs3[:, :, None, :] + xd3[:, None, :, :]
                     + cb[:, None, None, :])
    h1 = h1.reshape(ROWS_BLK, HID)
    h2 = jax.nn.relu(dot(h1, w2_ref[...]).astype(bf16)
                     + b2_ref[...])
    o = dot(h2, w3_ref[...])
    r = lax.broadcasted_iota(jnp.int32, (EDGES_PER_G, EDGES_PER_G), 0)
    cc = lax.broadcasted_iota(jnp.int32, (EDGES_PER_G, EDGES_PER_G), 1)
    Q = 0.5 * ((cc == (r % NN) * NN + r // NN).astype(f32)
               + (cc == r).astype(f32))
    cols = []
    for g in range(G_BLK):
        og = o[g * EDGES_PER_G:(g + 1) * EDGES_PER_G]
        cols.append(lax.dot_general(
            og, Q, (((0,), (0,)), ((), ())),
            preferred_element_type=f32))
    out_ref[...] = jnp.concatenate(cols, axis=1) + b3_ref[...]


def _tc_mlp(xid, xmult, z_graph, W1, b1, W2, b2, W3, b3, interpret=False):
    nb = z_graph.shape[0]
    grid = nb // G_BLK
    return pl.pallas_call(
        _tc_mlp_body,
        grid=(grid,),
        in_specs=[
            pl.BlockSpec((NODES_BLK, D), lambda i: (i, 0)),
            pl.BlockSpec((NODES_BLK, D), lambda i: (i, 0)),
            pl.BlockSpec((G_BLK, 32), lambda i: (i, 0)),
            pl.BlockSpec((FEAT, HID), lambda i: (0, 0)),
            pl.BlockSpec((1, HID), lambda i: (0, 0)),
            pl.BlockSpec((HID, FEAT), lambda i: (0, 0)),
            pl.BlockSpec((1, FEAT), lambda i: (0, 0)),
            pl.BlockSpec((FEAT, NUM_ATOMS), lambda i: (0, 0)),
            pl.BlockSpec((NUM_ATOMS, 1), lambda i: (0, 0)),
        ],
        out_specs=pl.BlockSpec((NUM_ATOMS, ROWS_BLK), lambda i: (0, i)),
        out_shape=jax.ShapeDtypeStruct(
            (NUM_ATOMS, nb * EDGES_PER_G), jnp.float32),
        compiler_params=pltpu.CompilerParams(
            dimension_semantics=("arbitrary",)),
        interpret=interpret,
    )(xid, xmult, z_graph, W1, b1, W2, b2, W3, b3)


def kernel(shape_node_idx, shape_node_mult, z_graph, id_table, mult_table,
           W1, b1, W2, b2, W3, b3):
    idx = shape_node_idx.astype(jnp.int32)
    mult = shape_node_mult.astype(jnp.int32)
    xid, xmult = _sc_gather(idx, mult, id_table, mult_table)
    out_t = _tc_mlp(xid, xmult, z_graph, W1,
                    b1.reshape(1, HID), W2.astype(jnp.bfloat16),
                    b2.astype(jnp.bfloat16).reshape(1, FEAT),
                    W3.astype(jnp.bfloat16), b3.reshape(NUM_ATOMS, 1))
    return jnp.transpose(out_t)


# speedup vs baseline: 1.2201x; 1.0383x over previous
import functools

import jax
import jax.numpy as jnp
import numpy as np
from jax import lax
from jax.experimental import pallas as pl
from jax.experimental.pallas import tpu as pltpu
from jax.experimental.pallas import tpu_sc as plsc

B = 512
NN = 16
N = B * NN
EDGES_PER_G = NN * NN
E = B * EDGES_PER_G
D = 16
HID = 256
FEAT = 128
NUM_ATOMS = 9

_NC = 2
_NS = 16
_NW = _NC * _NS
_BPW = N // _NW

G_BLK = 64
NODES_BLK = G_BLK * NN
ROWS_BLK = G_BLK * EDGES_PER_G
SEL_K = 2 * NODES_BLK + G_BLK


def _build_sel() -> np.ndarray:
    e = np.arange(ROWS_BLK)
    g, r = e // EDGES_PER_G, e % EDGES_PER_G
    a, b = r // NN, r % NN
    n = np.arange(NODES_BLK)
    sel_a = (n[None, :] == (NN * g + a)[:, None])
    sel_b = (n[None, :] == (NN * g + b)[:, None])
    sel_g = (np.arange(G_BLK)[None, :] == g[:, None])
    return np.concatenate([sel_a, sel_b, sel_g], axis=1).astype(np.float32)


_SEL = _build_sel()


def _make_sc_gather_body(bpw):
    def _sc_gather_body(idx_hbm, mult_hbm, idtab_hbm, multtab_hbm,
                        xid_out, xmult_out, idx_v, midx_v, rows_v, mrows_v,
                        sem, msem):
        wid = lax.axis_index("s") * _NC + lax.axis_index("c")
        base = wid * bpw
        pltpu.sync_copy(idx_hbm.at[pl.ds(base, bpw)], idx_v)
        pltpu.sync_copy(mult_hbm.at[pl.ds(base, bpw)], midx_v)
        c1 = pltpu.async_copy(idtab_hbm.at[idx_v], rows_v, sem)
        c2 = pltpu.async_copy(multtab_hbm.at[midx_v], mrows_v, msem)
        c1.wait()
        pltpu.sync_copy(rows_v, xid_out.at[pl.ds(base, bpw)])
        c2.wait()
        pltpu.sync_copy(mrows_v, xmult_out.at[pl.ds(base, bpw)])
    return _sc_gather_body


@jax.jit
def _sc_gather(idx, mult, id_table, mult_table):
    n = idx.shape[0]
    bpw = n // _NW
    mesh = plsc.VectorSubcoreMesh(core_axis_name="c", subcore_axis_name="s")
    fn = functools.partial(
        pl.kernel,
        out_type=[
            jax.ShapeDtypeStruct((n, D), jnp.float32),
            jax.ShapeDtypeStruct((n, D), jnp.float32),
        ],
        mesh=mesh,
        scratch_types=[
            pltpu.VMEM((bpw,), jnp.int32),
            pltpu.VMEM((bpw,), jnp.int32),
            pltpu.VMEM((bpw, D), jnp.float32),
            pltpu.VMEM((bpw, D), jnp.float32),
            pltpu.SemaphoreType.DMA,
            pltpu.SemaphoreType.DMA,
        ],
        compiler_params=pltpu.CompilerParams(use_tc_tiling_on_sc=False),
    )(_make_sc_gather_body(bpw))
    return fn(idx, mult, id_table, mult_table)


def _tc_mlp_body(xid_ref, xm_ref, z_ref, w1_ref, b1_ref, w2_ref,
                 b2_ref, w3_ref, b3_ref, out_ref):
    f32 = jnp.float32
    xid = xid_ref[...]
    xm = xm_ref[...]
    w1 = w1_ref[...]
    dot = functools.partial(jnp.dot, preferred_element_type=f32)
    xcat = jnp.concatenate([xid, xm], axis=1)
    wcat = jnp.concatenate([w1[0:32], w1[32:64], w1[96:128]], axis=1)
    big = dot(xcat, wcat)
    xs = big[:, 0:HID]
    xd = big[:, HID:2 * HID]
    xa = big[:, 2 * HID:3 * HID]
    gi = lax.broadcasted_iota(jnp.int32, (G_BLK, NODES_BLK), 0)
    ni = lax.broadcasted_iota(jnp.int32, (G_BLK, NODES_BLK), 1)
    ones_sel = (ni // NN == gi).astype(f32)
    c = (dot(ones_sel, xa) + dot(z_ref[...], w1[64:96])
         + b1_ref[...])
    bf16 = jnp.bfloat16
    xs3 = xs.astype(bf16).reshape(G_BLK, NN, HID)
    xd3 = xd.astype(bf16).reshape(G_BLK, NN, HID)
    cb = c.astype(bf16)
    h1 = jax.nn.relu(xs3[:, :, None, :] + xd3[:, None, :, :]
                     + cb[:, None, None, :])
    h1 = h1.reshape(ROWS_BLK, HID)
    h2 = jax.nn.relu(dot(h1, w2_ref[...]).astype(bf16)
                     + b2_ref[...])
    o = dot(h2, w3_ref[...])
    r = lax.broadcasted_iota(jnp.int32, (EDGES_PER_G, EDGES_PER_G), 0)
    cc = lax.broadcasted_iota(jnp.int32, (EDGES_PER_G, EDGES_PER_G), 1)
    Q = 0.5 * ((cc == (r % NN) * NN + r // NN).astype(f32)
               + (cc == r).astype(f32))
    cols = []
    for g in range(G_BLK):
        og = o[g * EDGES_PER_G:(g + 1) * EDGES_PER_G]
        cols.append(lax.dot_general(
            og, Q, (((0,), (0,)), ((), ())),
            preferred_element_type=f32))
    out_ref[...] = jnp.concatenate(cols, axis=1) + b3_ref[...]


def _tc_mlp(xid, xmult, z_graph, W1, b1, W2, b2, W3, b3, interpret=False):
    nb = z_graph.shape[0]
    grid = nb // G_BLK
    return pl.pallas_call(
        _tc_mlp_body,
        grid=(grid,),
        in_specs=[
            pl.BlockSpec((NODES_BLK, D), lambda i: (i, 0)),
            pl.BlockSpec((NODES_BLK, D), lambda i: (i, 0)),
            pl.BlockSpec((G_BLK, 32), lambda i: (i, 0)),
            pl.BlockSpec((FEAT, HID), lambda i: (0, 0)),
            pl.BlockSpec((1, HID), lambda i: (0, 0)),
            pl.BlockSpec((HID, FEAT), lambda i: (0, 0)),
            pl.BlockSpec((1, FEAT), lambda i: (0, 0)),
            pl.BlockSpec((FEAT, NUM_ATOMS), lambda i: (0, 0)),
            pl.BlockSpec((NUM_ATOMS, 1), lambda i: (0, 0)),
        ],
        out_specs=pl.BlockSpec((NUM_ATOMS, ROWS_BLK), lambda i: (0, i)),
        out_shape=jax.ShapeDtypeStruct(
            (NUM_ATOMS, nb * EDGES_PER_G), jnp.float32),
        compiler_params=pltpu.CompilerParams(
            dimension_semantics=("arbitrary",)),
        interpret=interpret,
    )(xid, xmult, z_graph, W1, b1, W2, b2, W3, b3)


def kernel(shape_node_idx, shape_node_mult, z_graph, id_table, mult_table,
           W1, b1, W2, b2, W3, b3):
    idx = shape_node_idx.astype(jnp.int32)
    mult = shape_node_mult.astype(jnp.int32)
    xid, xmult = _sc_gather(idx, mult, id_table, mult_table)
    out_t = _tc_mlp(xid, xmult, z_graph, W1,
                    b1.reshape(1, HID), W2.astype(jnp.bfloat16),
                    b2.astype(jnp.bfloat16).reshape(1, FEAT),
                    W3.astype(jnp.bfloat16), b3.reshape(NUM_ATOMS, 1))
    return jnp.transpose(out_t)


# speedup vs baseline: 1.2340x; 1.0114x over previous
import functools

import jax
import jax.numpy as jnp
import numpy as np
from jax import lax
from jax.experimental import pallas as pl
from jax.experimental.pallas import tpu as pltpu
from jax.experimental.pallas import tpu_sc as plsc

B = 512
NN = 16
N = B * NN
EDGES_PER_G = NN * NN
E = B * EDGES_PER_G
D = 16
HID = 256
FEAT = 128
NUM_ATOMS = 9

_NC = 2
_NS = 16
_NW = _NC * _NS
_BPW = N // _NW

G_BLK = 128
NODES_BLK = G_BLK * NN
ROWS_BLK = G_BLK * EDGES_PER_G
SEL_K = 2 * NODES_BLK + G_BLK


def _build_sel() -> np.ndarray:
    e = np.arange(ROWS_BLK)
    g, r = e // EDGES_PER_G, e % EDGES_PER_G
    a, b = r // NN, r % NN
    n = np.arange(NODES_BLK)
    sel_a = (n[None, :] == (NN * g + a)[:, None])
    sel_b = (n[None, :] == (NN * g + b)[:, None])
    sel_g = (np.arange(G_BLK)[None, :] == g[:, None])
    return np.concatenate([sel_a, sel_b, sel_g], axis=1).astype(np.float32)


_SEL = _build_sel()


def _make_sc_gather_body(bpw):
    def _sc_gather_body(idx_hbm, mult_hbm, idtab_hbm, multtab_hbm,
                        xid_out, xmult_out, idx_v, midx_v, rows_v, mrows_v,
                        sem, msem):
        wid = lax.axis_index("s") * _NC + lax.axis_index("c")
        base = wid * bpw
        pltpu.sync_copy(idx_hbm.at[pl.ds(base, bpw)], idx_v)
        pltpu.sync_copy(mult_hbm.at[pl.ds(base, bpw)], midx_v)
        c1 = pltpu.async_copy(idtab_hbm.at[idx_v], rows_v, sem)
        c2 = pltpu.async_copy(multtab_hbm.at[midx_v], mrows_v, msem)
        c1.wait()
        pltpu.sync_copy(rows_v, xid_out.at[pl.ds(base, bpw)])
        c2.wait()
        pltpu.sync_copy(mrows_v, xmult_out.at[pl.ds(base, bpw)])
    return _sc_gather_body


@jax.jit
def _sc_gather(idx, mult, id_table, mult_table):
    n = idx.shape[0]
    bpw = n // _NW
    mesh = plsc.VectorSubcoreMesh(core_axis_name="c", subcore_axis_name="s")
    fn = functools.partial(
        pl.kernel,
        out_type=[
            jax.ShapeDtypeStruct((n, D), jnp.float32),
            jax.ShapeDtypeStruct((n, D), jnp.float32),
        ],
        mesh=mesh,
        scratch_types=[
            pltpu.VMEM((bpw,), jnp.int32),
            pltpu.VMEM((bpw,), jnp.int32),
            pltpu.VMEM((bpw, D), jnp.float32),
            pltpu.VMEM((bpw, D), jnp.float32),
            pltpu.SemaphoreType.DMA,
            pltpu.SemaphoreType.DMA,
        ],
        compiler_params=pltpu.CompilerParams(use_tc_tiling_on_sc=False),
    )(_make_sc_gather_body(bpw))
    return fn(idx, mult, id_table, mult_table)


def _tc_mlp_body(xid_ref, xm_ref, z_ref, w1_ref, b1_ref, w2_ref,
                 b2_ref, w3_ref, b3_ref, out_ref):
    f32 = jnp.float32
    xid = xid_ref[...]
    xm = xm_ref[...]
    w1 = w1_ref[...]
    dot = functools.partial(jnp.dot, preferred_element_type=f32)
    xcat = jnp.concatenate([xid, xm], axis=1)
    wcat = jnp.concatenate([w1[0:32], w1[32:64], w1[96:128]], axis=1)
    big = dot(xcat, wcat)
    xs = big[:, 0:HID]
    xd = big[:, HID:2 * HID]
    xa = big[:, 2 * HID:3 * HID]
    gi = lax.broadcasted_iota(jnp.int32, (G_BLK, NODES_BLK), 0)
    ni = lax.broadcasted_iota(jnp.int32, (G_BLK, NODES_BLK), 1)
    ones_sel = (ni // NN == gi).astype(f32)
    c = (dot(ones_sel, xa) + dot(z_ref[...], w1[64:96])
         + b1_ref[...])
    bf16 = jnp.bfloat16
    xs3 = xs.astype(bf16).reshape(G_BLK, NN, HID)
    xd3 = xd.astype(bf16).reshape(G_BLK, NN, HID)
    cb = c.astype(bf16)
    h1 = jax.nn.relu(xs3[:, :, None, :] + xd3[:, None, :, :]
                     + cb[:, None, None, :])
    h1 = h1.reshape(ROWS_BLK, HID)
    h2 = jax.nn.relu(dot(h1, w2_ref[...]).astype(bf16)
                     + b2_ref[...])
    o = dot(h2, w3_ref[...])
    r = lax.broadcasted_iota(jnp.int32, (EDGES_PER_G, EDGES_PER_G), 0)
    cc = lax.broadcasted_iota(jnp.int32, (EDGES_PER_G, EDGES_PER_G), 1)
    Q = 0.5 * ((cc == (r % NN) * NN + r // NN).astype(f32)
               + (cc == r).astype(f32))
    cols = []
    for g in range(G_BLK):
        og = o[g * EDGES_PER_G:(g + 1) * EDGES_PER_G]
        cols.append(lax.dot_general(
            og, Q, (((0,), (0,)), ((), ())),
            preferred_element_type=f32))
    out_ref[...] = jnp.concatenate(cols, axis=1) + b3_ref[...]


def _tc_mlp(xid, xmult, z_graph, W1, b1, W2, b2, W3, b3, interpret=False):
    nb = z_graph.shape[0]
    grid = nb // G_BLK
    return pl.pallas_call(
        _tc_mlp_body,
        grid=(grid,),
        in_specs=[
            pl.BlockSpec((NODES_BLK, D), lambda i: (i, 0)),
            pl.BlockSpec((NODES_BLK, D), lambda i: (i, 0)),
            pl.BlockSpec((G_BLK, 32), lambda i: (i, 0)),
            pl.BlockSpec((FEAT, HID), lambda i: (0, 0)),
            pl.BlockSpec((1, HID), lambda i: (0, 0)),
            pl.BlockSpec((HID, FEAT), lambda i: (0, 0)),
            pl.BlockSpec((1, FEAT), lambda i: (0, 0)),
            pl.BlockSpec((FEAT, NUM_ATOMS), lambda i: (0, 0)),
            pl.BlockSpec((NUM_ATOMS, 1), lambda i: (0, 0)),
        ],
        out_specs=pl.BlockSpec((NUM_ATOMS, ROWS_BLK), lambda i: (0, i)),
        out_shape=jax.ShapeDtypeStruct(
            (NUM_ATOMS, nb * EDGES_PER_G), jnp.float32),
        compiler_params=pltpu.CompilerParams(
            dimension_semantics=("arbitrary",)),
        interpret=interpret,
    )(xid, xmult, z_graph, W1, b1, W2, b2, W3, b3)


def kernel(shape_node_idx, shape_node_mult, z_graph, id_table, mult_table,
           W1, b1, W2, b2, W3, b3):
    idx = shape_node_idx.astype(jnp.int32)
    mult = shape_node_mult.astype(jnp.int32)
    xid, xmult = _sc_gather(idx, mult, id_table, mult_table)
    out_t = _tc_mlp(xid, xmult, z_graph, W1,
                    b1.reshape(1, HID), W2.astype(jnp.bfloat16),
                    b2.astype(jnp.bfloat16).reshape(1, FEAT),
                    W3.astype(jnp.bfloat16), b3.reshape(NUM_ATOMS, 1))
    return jnp.transpose(out_t)


# speedup vs baseline: 1.2679x; 1.0275x over previous
import functools

import jax
import jax.numpy as jnp
import numpy as np
from jax import lax
from jax.experimental import pallas as pl
from jax.experimental.pallas import tpu as pltpu
from jax.experimental.pallas import tpu_sc as plsc

B = 512
NN = 16
N = B * NN
EDGES_PER_G = NN * NN
E = B * EDGES_PER_G
D = 16
HID = 256
FEAT = 128
NUM_ATOMS = 9

_NC = 2
_NS = 16
_NW = _NC * _NS
_BPW = N // _NW

G_BLK = 128
NODES_BLK = G_BLK * NN
ROWS_BLK = G_BLK * EDGES_PER_G
SEL_K = 2 * NODES_BLK + G_BLK


def _build_sel() -> np.ndarray:
    e = np.arange(ROWS_BLK)
    g, r = e // EDGES_PER_G, e % EDGES_PER_G
    a, b = r // NN, r % NN
    n = np.arange(NODES_BLK)
    sel_a = (n[None, :] == (NN * g + a)[:, None])
    sel_b = (n[None, :] == (NN * g + b)[:, None])
    sel_g = (np.arange(G_BLK)[None, :] == g[:, None])
    return np.concatenate([sel_a, sel_b, sel_g], axis=1).astype(np.float32)


_SEL = _build_sel()


def _make_sc_gather_body(bpw):
    def _sc_gather_body(idx_hbm, mult_hbm, idtab_hbm, multtab_hbm,
                        xid_out, xmult_out, idx_v, midx_v, rows_v, mrows_v,
                        sem, msem):
        wid = lax.axis_index("s") * _NC + lax.axis_index("c")
        base = wid * bpw
        row0 = (wid // 8) * bpw
        col0 = (wid % 8) * D
        pltpu.sync_copy(idx_hbm.at[pl.ds(base, bpw)], idx_v)
        pltpu.sync_copy(mult_hbm.at[pl.ds(base, bpw)], midx_v)
        c1 = pltpu.async_copy(idtab_hbm.at[idx_v], rows_v, sem)
        c2 = pltpu.async_copy(multtab_hbm.at[midx_v], mrows_v, msem)
        c1.wait()
        pltpu.sync_copy(rows_v, xid_out.at[pl.ds(row0, bpw), pl.ds(col0, D)])
        c2.wait()
        pltpu.sync_copy(mrows_v,
                        xmult_out.at[pl.ds(row0, bpw), pl.ds(col0, D)])
    return _sc_gather_body


@jax.jit
def _sc_gather(idx, mult, id_table, mult_table):
    n = idx.shape[0]
    bpw = n // _NW
    mesh = plsc.VectorSubcoreMesh(core_axis_name="c", subcore_axis_name="s")
    fn = functools.partial(
        pl.kernel,
        out_type=[
            jax.ShapeDtypeStruct((n * D // 128, 128), jnp.float32),
            jax.ShapeDtypeStruct((n * D // 128, 128), jnp.float32),
        ],
        mesh=mesh,
        scratch_types=[
            pltpu.VMEM((bpw,), jnp.int32),
            pltpu.VMEM((bpw,), jnp.int32),
            pltpu.VMEM((bpw, D), jnp.float32),
            pltpu.VMEM((bpw, D), jnp.float32),
            pltpu.SemaphoreType.DMA,
            pltpu.SemaphoreType.DMA,
        ],
        compiler_params=pltpu.CompilerParams(use_tc_tiling_on_sc=False),
    )(_make_sc_gather_body(bpw))
    return fn(idx, mult, id_table, mult_table)


def _tc_mlp_body(xid_ref, xm_ref, z_ref, w1_ref, b1_ref, w2_ref,
                 b2_ref, w3_ref, b3_ref, out_ref):
    f32 = jnp.float32
    xid_p = xid_ref[...]
    xm_p = xm_ref[...]
    xid = jnp.concatenate([xid_p[:, 16 * j:16 * (j + 1)] for j in range(8)],
                          axis=0)
    xm = jnp.concatenate([xm_p[:, 16 * j:16 * (j + 1)] for j in range(8)],
                         axis=0)
    w1 = w1_ref[...]
    dot = functools.partial(jnp.dot, preferred_element_type=f32)
    xcat = jnp.concatenate([xid, xm], axis=1)
    wcat = jnp.concatenate([w1[0:32], w1[32:64], w1[96:128]], axis=1)
    big = dot(xcat, wcat)
    xs = big[:, 0:HID]
    xd = big[:, HID:2 * HID]
    xa = big[:, 2 * HID:3 * HID]
    gi = lax.broadcasted_iota(jnp.int32, (G_BLK, NODES_BLK), 0)
    ni = lax.broadcasted_iota(jnp.int32, (G_BLK, NODES_BLK), 1)
    ones_sel = (ni // NN == gi).astype(f32)
    c = (dot(ones_sel, xa) + dot(z_ref[...], w1[64:96])
         + b1_ref[...])
    bf16 = jnp.bfloat16
    xs3 = xs.astype(bf16).reshape(G_BLK, NN, HID)
    xd3 = xd.astype(bf16).reshape(G_BLK, NN, HID)
    cb = c.astype(bf16)
    h1 = jax.nn.relu(xs3[:, :, None, :] + xd3[:, None, :, :]
                     + cb[:, None, None, :])
    h1 = h1.reshape(ROWS_BLK, HID)
    h2 = jax.nn.relu(dot(h1, w2_ref[...]).astype(bf16)
                     + b2_ref[...])
    o = dot(h2, w3_ref[...])
    r = lax.broadcasted_iota(jnp.int32, (EDGES_PER_G, EDGES_PER_G), 0)
    cc = lax.broadcasted_iota(jnp.int32, (EDGES_PER_G, EDGES_PER_G), 1)
    Q = 0.5 * ((cc == (r % NN) * NN + r // NN).astype(f32)
               + (cc == r).astype(f32))
    cols = []
    for g in range(G_BLK):
        og = o[g * EDGES_PER_G:(g + 1) * EDGES_PER_G]
        cols.append(lax.dot_general(
            og, Q, (((0,), (0,)), ((), ())),
            preferred_element_type=f32))
    out_ref[...] = jnp.concatenate(cols, axis=1) + b3_ref[...]


def _tc_mlp(xid, xmult, z_graph, W1, b1, W2, b2, W3, b3, interpret=False):
    nb = z_graph.shape[0]
    grid = nb // G_BLK
    return pl.pallas_call(
        _tc_mlp_body,
        grid=(grid,),
        in_specs=[
            pl.BlockSpec((NODES_BLK * D // 128, 128), lambda i: (i, 0)),
            pl.BlockSpec((NODES_BLK * D // 128, 128), lambda i: (i, 0)),
            pl.BlockSpec((G_BLK, 32), lambda i: (i, 0)),
            pl.BlockSpec((FEAT, HID), lambda i: (0, 0)),
            pl.BlockSpec((1, HID), lambda i: (0, 0)),
            pl.BlockSpec((HID, FEAT), lambda i: (0, 0)),
            pl.BlockSpec((1, FEAT), lambda i: (0, 0)),
            pl.BlockSpec((FEAT, NUM_ATOMS), lambda i: (0, 0)),
            pl.BlockSpec((NUM_ATOMS, 1), lambda i: (0, 0)),
        ],
        out_specs=pl.BlockSpec((NUM_ATOMS, ROWS_BLK), lambda i: (0, i)),
        out_shape=jax.ShapeDtypeStruct(
            (NUM_ATOMS, nb * EDGES_PER_G), jnp.float32),
        compiler_params=pltpu.CompilerParams(
            dimension_semantics=("arbitrary",)),
        interpret=interpret,
    )(xid, xmult, z_graph, W1, b1, W2, b2, W3, b3)


def kernel(shape_node_idx, shape_node_mult, z_graph, id_table, mult_table,
           W1, b1, W2, b2, W3, b3):
    idx = shape_node_idx.astype(jnp.int32)
    mult = shape_node_mult.astype(jnp.int32)
    xid, xmult = _sc_gather(idx, mult, id_table, mult_table)
    out_t = _tc_mlp(xid, xmult, z_graph, W1,
                    b1.reshape(1, HID), W2.astype(jnp.bfloat16),
                    b2.astype(jnp.bfloat16).reshape(1, FEAT),
                    W3.astype(jnp.bfloat16), b3.reshape(NUM_ATOMS, 1))
    return jnp.transpose(out_t)
